# baseline (speedup 1.0000x reference)
import functools

import jax
import jax.numpy as jnp
from jax import lax
from jax.experimental import pallas as pl
from jax.experimental.pallas import tpu as pltpu
from jax.experimental.pallas import tpu_sc as plsc

_E = 320000
_N = 10000
_D = 128
_GD = 16

_NW = 32
_CSZ = 80
_CH = 125
_SLAB = _CH * _CSZ
_NH = 16
_H = _NH * _SLAB
_N_PAD = 10240
_RPT = _N_PAD // 16


def _mesh():
    return plsc.VectorSubcoreMesh(core_axis_name="c", subcore_axis_name="s",
                                  num_cores=2, num_subcores=16)


def _gather_pair(table_a, table_b, idx_a, idx_b):
    M = 2
    G = (_CH - 1) // M

    @functools.partial(
        pl.kernel,
        out_type=(jax.ShapeDtypeStruct((_H, _D), jnp.float32),
                  jax.ShapeDtypeStruct((_H, _D), jnp.float32)),
        mesh=_mesh(),
        scratch_types=[pltpu.VMEM((_CH, _CSZ), jnp.int32),
                       pltpu.VMEM_SHARED((_N_PAD, _D), jnp.float32)]
                      + [pltpu.VMEM((_CSZ, _D), jnp.float32)] * M
                      + [pltpu.SemaphoreType.DMA] * (2 * M),
    )
    def gk(ta, tb, ia, ib, oa, ob, idx_v, tab_s, *bufs_and_sems):
        rows = bufs_and_sems[:M]
        gsem = bufs_and_sems[M:2 * M]
        wsem = bufs_and_sems[2 * M:]
        c = lax.axis_index("c")
        s = lax.axis_index("s")
        row0 = s * _RPT

        @pl.when(c == 0)
        def _():
            pltpu.sync_copy(ta.at[pl.ds(row0, _RPT)], tab_s.at[pl.ds(row0, _RPT)])

        @pl.when(c == 1)
        def _():
            pltpu.sync_copy(tb.at[pl.ds(row0, _RPT)], tab_s.at[pl.ds(row0, _RPT)])

        plsc.subcore_barrier()

        def run(idx_hbm, out):
            base = s * _SLAB
            pltpu.sync_copy(idx_hbm.at[s], idx_v)
            for b in range(M):
                pltpu.async_copy(tab_s.at[idx_v.at[b]], rows[b], gsem[b])

            def group(g, carry):
                for b in range(M):
                    j = g * M + b
                    pltpu.make_async_copy(tab_s.at[idx_v.at[0]], rows[b],
                                          gsem[b]).wait()
                    pltpu.async_copy(rows[b],
                                     out.at[pl.ds(base + j * _CSZ, _CSZ)],
                                     wsem[b])

                @pl.when(g < G - 1)
                def _():
                    for b in range(M):
                        jn = (g + 1) * M + b
                        pltpu.make_async_copy(
                            rows[b], out.at[pl.ds(0, _CSZ)], wsem[b]).wait()
                        pltpu.async_copy(tab_s.at[idx_v.at[jn]], rows[b],
                                         gsem[b])

                return carry

            lax.fori_loop(0, G, group, 0)
            for b in range(M):
                pltpu.make_async_copy(rows[b], out.at[pl.ds(0, _CSZ)],
                                      wsem[b]).wait()
            j = _CH - 1
            pltpu.async_copy(tab_s.at[idx_v.at[j]], rows[0], gsem[0])
            pltpu.make_async_copy(tab_s.at[idx_v.at[0]], rows[0], gsem[0]).wait()
            pltpu.async_copy(rows[0], out.at[pl.ds(base + j * _CSZ, _CSZ)],
                             wsem[0])
            pltpu.make_async_copy(rows[0], out.at[pl.ds(0, _CSZ)],
                                  wsem[0]).wait()

        @pl.when(c == 0)
        def _():
            run(ia, oa)

        @pl.when(c == 1)
        def _():
            run(ib, ob)

    return gk(table_a, table_b, idx_a, idx_b)


def _scatter_pair(msg_l, idx_r, msg_r, idx_l, zeros):
    M = 2
    G = (_CH - 1) // M

    @functools.partial(
        pl.kernel,
        out_type=(jax.ShapeDtypeStruct((_N_PAD, _D), jnp.float32),
                  jax.ShapeDtypeStruct((_N_PAD, _D), jnp.float32)),
        mesh=_mesh(),
        scratch_types=[pltpu.VMEM((_CH, _CSZ), jnp.int32),
                       pltpu.VMEM_SHARED((_N_PAD, _D), jnp.float32)]
                      + [pltpu.VMEM((_CSZ, _D), jnp.float32)] * M
                      + [pltpu.SemaphoreType.DMA] * (2 * M),
    )
    def sk(ml, ir, mr, il, z, ol, orr, idx_v, acc, *bufs_and_sems):
        rows = bufs_and_sems[:M]
        rsem = bufs_and_sems[M:2 * M]
        ssem = bufs_and_sems[2 * M:]
        c = lax.axis_index("c")
        s = lax.axis_index("s")
        row0 = s * _RPT
        pltpu.sync_copy(z.at[pl.ds(row0, _RPT)], acc.at[pl.ds(row0, _RPT)])
        plsc.subcore_barrier()

        def run(msg, idx_hbm):
            base = s * _SLAB
            pltpu.sync_copy(idx_hbm.at[s], idx_v)
            for b in range(M):
                pltpu.async_copy(msg.at[pl.ds(base + b * _CSZ, _CSZ)],
                                 rows[b], rsem[b])

            def group(g, carry):
                for b in range(M):
                    j = g * M + b
                    pltpu.make_async_copy(msg.at[pl.ds(0, _CSZ)], rows[b],
                                          rsem[b]).wait()
                    pltpu.async_copy(rows[b], acc.at[idx_v.at[j]], ssem[b],
                                     add=True)

                @pl.when(g < G - 1)
                def _():
                    for b in range(M):
                        jn = (g + 1) * M + b
                        pltpu.make_async_copy(
                            rows[b], acc.at[idx_v.at[0]], ssem[b]).wait()
                        pltpu.async_copy(
                            msg.at[pl.ds(base + jn * _CSZ, _CSZ)],
                            rows[b], rsem[b])

                return carry

            lax.fori_loop(0, G, group, 0)
            for b in range(M):
                pltpu.make_async_copy(rows[b], acc.at[idx_v.at[0]],
                                      ssem[b]).wait()
            j = _CH - 1
            pltpu.async_copy(msg.at[pl.ds(base + j * _CSZ, _CSZ)], rows[0],
                             rsem[0])
            pltpu.make_async_copy(msg.at[pl.ds(0, _CSZ)], rows[0],
                                  rsem[0]).wait()
            pltpu.async_copy(rows[0], acc.at[idx_v.at[j]], ssem[0], add=True)
            pltpu.make_async_copy(rows[0], acc.at[idx_v.at[0]],
                                  ssem[0]).wait()

        @pl.when(c == 0)
        def _():
            run(ml, ir)

        @pl.when(c == 1)
        def _():
            run(mr, il)

        plsc.subcore_barrier()

        @pl.when(c == 0)
        def _():
            pltpu.sync_copy(acc.at[pl.ds(row0, _RPT)], ol.at[pl.ds(row0, _RPT)])

        @pl.when(c == 1)
        def _():
            pltpu.sync_copy(acc.at[pl.ds(row0, _RPT)], orr.at[pl.ds(row0, _RPT)])

    return sk(msg_l, idx_r, msg_r, idx_l, zeros)


def _ln_k(x, g, b):
    m = jnp.mean(x, -1, keepdims=True)
    d = x - m
    v = jnp.mean(d * d, -1, keepdims=True)
    return d * lax.rsqrt(v + 1e-5) * g + b


def _side(hb, hn, ex, W):
    (bw, nw, i0b, ig, ibb, i1w, i1b,
     gwb, gwn, gwe, g0b, gg, gbb, g1w, g1b) = [w[...] for w in W]
    h = jnp.dot(hb, bw) + jnp.dot(hn, nw) + i0b
    h = jnp.maximum(_ln_k(h, ig, ibb), 0.0)
    io = jnp.dot(h, i1w) + i1b
    g = (jnp.dot(hb, gwb) + jnp.dot(hn, gwn)
         + lax.dot_general(ex, gwe, (((0,), (0,)), ((), ()))) + g0b)
    g = jnp.maximum(_ln_k(g, gg, gbb), 0.0)
    go = jnp.dot(g, g1w) + g1b
    return io * jax.nn.sigmoid(go)


def _full_spec(w):
    return pl.BlockSpec(w.shape, lambda i, n=len(w.shape): (0,) * n)


def _edge_call(hb, hl, hr, ex, ws, off):
    BE = 1280
    grid = _H // BE

    def body(hb_r, hl_r, hr_r, ex_r, *rest):
        wrefs = rest[:30]
        ml_r, mr_r = rest[30:]
        hb_v = hb_r[...]
        ex_v = ex_r[...]
        ml_r[...] = _side(hb_v, hl_r[...], ex_v, wrefs[:15])
        mr_r[...] = _side(hb_v, hr_r[...], ex_v, wrefs[15:])

    obk = lambda w: pl.BlockSpec((BE, w), lambda i: (i + off, 0))
    blk = lambda w: pl.BlockSpec((BE, w), lambda i: (i, 0))
    ebk = pl.BlockSpec((_GD, BE), lambda i: (0, i + off))
    return pl.pallas_call(
        body,
        grid=(grid,),
        in_specs=[obk(_D), blk(_D), blk(_D), ebk] + [_full_spec(w) for w in ws],
        out_specs=[blk(_D), blk(_D)],
        out_shape=[jax.ShapeDtypeStruct((_H, _D), jnp.float32),
                   jax.ShapeDtypeStruct((_H, _D), jnp.float32)],
    )(hb, hl, hr, ex, *ws)


def _table_call(accs, hn, wml, wnl, bl, wmr, wnr, br):
    def body(al0_r, al1_r, ar0_r, ar1_r, hn_r, wml_r, wnl_r, bl_r,
             wmr_r, wnr_r, br_r, a_r, b_r):
        hn_v = hn_r[...]
        al = al0_r[...] + al1_r[...]
        ar = ar0_r[...] + ar1_r[...]
        a_r[...] = jnp.dot(al, wml_r[...]) + jnp.dot(hn_v, wnl_r[...]) + bl_r[...]
        b_r[...] = jnp.dot(ar, wmr_r[...]) + jnp.dot(hn_v, wnr_r[...]) + br_r[...]

    return pl.pallas_call(
        body,
        out_shape=[jax.ShapeDtypeStruct((_N_PAD, _D), jnp.float32),
                   jax.ShapeDtypeStruct((_N_PAD, _D), jnp.float32)],
    )(*accs, hn, wml, wnl, bl, wmr, wnr, br)


def _final_half(hb, al, br, ws, off, prev):
    BE = 1280
    grid = _H // BE

    def body(hb_r, al_r, br_r, wself, bself,
             og, obb, o1w, o1b, lng, lnb, *rest):
        out_r = rest[-1]
        hb_v = hb_r[...]
        h = al_r[...] + br_r[...] + jnp.dot(hb_v, wself[...]) + bself[...]
        h = jnp.maximum(_ln_k(h, og[...], obb[...]), 0.0)
        h = jnp.dot(h, o1w[...]) + o1b[...]
        out_r[...] = _ln_k(h + hb_v, lng[...], lnb[...])

    obk = pl.BlockSpec((BE, _D), lambda i: (i + off, 0))
    blk = pl.BlockSpec((BE, _D), lambda i: (i, 0))
    in_specs = [obk, blk, blk] + [_full_spec(w) for w in ws]
    args = [hb, al, br] + list(ws)
    kwargs = {}
    if prev is not None:
        in_specs.append(pl.BlockSpec(memory_space=pltpu.MemorySpace.HBM))
        args.append(prev)
        kwargs["input_output_aliases"] = {len(args) - 1: 0}
    return pl.pallas_call(
        body,
        grid=(grid,),
        in_specs=in_specs,
        out_specs=obk,
        out_shape=jax.ShapeDtypeStruct((_E, _D), jnp.float32),
        **kwargs,
    )(*args)


def _side_weights(p):
    it = p["inter"]
    g = p["gate"]
    gw = g["l0"]["w"]
    i0w = it["l0"]["w"]
    r = lambda x: x.reshape(1, -1)
    return [
        p["bond_w"] @ i0w, p["node_w"] @ i0w,
        r(it["l0"]["b"]), r(it["ln0"]["g"]), r(it["ln0"]["b"]),
        it["l1"]["w"], r(it["l1"]["b"]),
        gw[:_D], gw[_D:2 * _D], gw[2 * _D:],
        r(g["l0"]["b"]), r(g["ln0"]["g"]), r(g["ln0"]["b"]),
        g["l1"]["w"], r(g["l1"]["b"]),
    ]


def kernel(h_bond, bond_index, h_node, bond_extra, params):
    left = bond_index[0].astype(jnp.int32)
    right = bond_index[1].astype(jnp.int32)
    idx_l = left.reshape(2, _NH, _CH, _CSZ)
    idx_r = right.reshape(2, _NH, _CH, _CSZ)

    hn_p = jnp.pad(h_node, ((0, _N_PAD - _N), (0, 0)))
    zeros = jnp.zeros((_N_PAD, _D), jnp.float32)

    ws = _side_weights(params["ffn_l"]) + _side_weights(params["ffn_r"])
    hl0, hr0 = _gather_pair(hn_p, hn_p, idx_l[0], idx_r[0])
    hl1, hr1 = _gather_pair(hn_p, hn_p, idx_l[1], idx_r[1])
    ex_t = bond_extra.T
    ml0, mr0 = _edge_call(h_bond, hl0, hr0, ex_t, ws, 0)
    ml1, mr1 = _edge_call(h_bond, hl1, hr1, ex_t, ws, _H // 1280)
    accl0, accr0 = _scatter_pair(ml0, idx_r[0], mr0, idx_l[0], zeros)
    accl1, accr1 = _scatter_pair(ml1, idx_r[1], mr1, idx_l[1], zeros)

    r = lambda x: x.reshape(1, -1)
    fo = params["out"]
    o0w = fo["l0"]["w"]
    A, B = _table_call(
        (accl0, accl1, accr0, accr1), hn_p,
        params["msg_l"]["w"] @ o0w, params["node_l"]["w"] @ o0w,
        r((params["msg_l"]["b"] + params["node_l"]["b"]) @ o0w),
        params["msg_r"]["w"] @ o0w, params["node_r"]["w"] @ o0w,
        r((params["msg_r"]["b"] + params["node_r"]["b"]) @ o0w))

    al0, br0 = _gather_pair(A, B, idx_l[0], idx_r[0])
    al1, br1 = _gather_pair(A, B, idx_l[1], idx_r[1])

    fws = [params["self"]["w"] @ o0w,
           r(params["self"]["b"] @ o0w + fo["l0"]["b"]),
           r(fo["ln0"]["g"]), r(fo["ln0"]["b"]),
           fo["l1"]["w"], r(fo["l1"]["b"]),
           r(params["ln"]["g"]), r(params["ln"]["b"])]
    out0 = _final_half(h_bond, al0, br0, fws, 0, None)
    return _final_half(h_bond, al1, br1, fws, _H // 1280, out0)


# speedup vs baseline: 1.1022x; 1.1022x over previous
import functools

import jax
import jax.numpy as jnp
from jax import lax
from jax.experimental import pallas as pl
from jax.experimental.pallas import tpu as pltpu
from jax.experimental.pallas import tpu_sc as plsc

_E = 320000
_N = 10000
_D = 128
_GD = 16

_NW = 32
_CSZ = 80
_CH = 125
_SLAB = _CH * _CSZ
_NH = 16
_H = _NH * _SLAB
_N_PAD = 10240
_RPT = _N_PAD // 16


def _mesh():
    return plsc.VectorSubcoreMesh(core_axis_name="c", subcore_axis_name="s",
                                  num_cores=2, num_subcores=16)


def _gather_pair(table_a, table_b, idx_a, idx_b):
    M = 2
    G = (_CH - 1) // M

    @functools.partial(
        pl.kernel,
        out_type=(jax.ShapeDtypeStruct((_H, _D), jnp.float32),
                  jax.ShapeDtypeStruct((_H, _D), jnp.float32)),
        mesh=_mesh(),
        scratch_types=[pltpu.VMEM((_CH, _CSZ), jnp.int32),
                       pltpu.VMEM_SHARED((_N_PAD, _D), jnp.float32)]
                      + [pltpu.VMEM((_CSZ, _D), jnp.float32)] * M
                      + [pltpu.SemaphoreType.DMA] * (2 * M),
    )
    def gk(ta, tb, ia, ib, oa, ob, idx_v, tab_s, *bufs_and_sems):
        rows = bufs_and_sems[:M]
        gsem = bufs_and_sems[M:2 * M]
        wsem = bufs_and_sems[2 * M:]
        c = lax.axis_index("c")
        s = lax.axis_index("s")
        row0 = s * _RPT

        @pl.when(c == 0)
        def _():
            pltpu.sync_copy(ta.at[pl.ds(row0, _RPT)], tab_s.at[pl.ds(row0, _RPT)])

        @pl.when(c == 1)
        def _():
            pltpu.sync_copy(tb.at[pl.ds(row0, _RPT)], tab_s.at[pl.ds(row0, _RPT)])

        plsc.subcore_barrier()

        def run(idx_hbm, out):
            base = s * _SLAB
            pltpu.sync_copy(idx_hbm.at[s], idx_v)
            for b in range(M):
                pltpu.async_copy(tab_s.at[idx_v.at[b]], rows[b], gsem[b])

            def group(g, carry):
                for b in range(M):
                    j = g * M + b
                    pltpu.make_async_copy(tab_s.at[idx_v.at[0]], rows[b],
                                          gsem[b]).wait()
                    pltpu.async_copy(rows[b],
                                     out.at[pl.ds(base + j * _CSZ, _CSZ)],
                                     wsem[b])

                @pl.when(g < G - 1)
                def _():
                    for b in range(M):
                        jn = (g + 1) * M + b
                        pltpu.make_async_copy(
                            rows[b], out.at[pl.ds(0, _CSZ)], wsem[b]).wait()
                        pltpu.async_copy(tab_s.at[idx_v.at[jn]], rows[b],
                                         gsem[b])

                return carry

            lax.fori_loop(0, G, group, 0)
            for b in range(M):
                pltpu.make_async_copy(rows[b], out.at[pl.ds(0, _CSZ)],
                                      wsem[b]).wait()
            j = _CH - 1
            pltpu.async_copy(tab_s.at[idx_v.at[j]], rows[0], gsem[0])
            pltpu.make_async_copy(tab_s.at[idx_v.at[0]], rows[0], gsem[0]).wait()
            pltpu.async_copy(rows[0], out.at[pl.ds(base + j * _CSZ, _CSZ)],
                             wsem[0])
            pltpu.make_async_copy(rows[0], out.at[pl.ds(0, _CSZ)],
                                  wsem[0]).wait()

        @pl.when(c == 0)
        def _():
            run(ia, oa)

        @pl.when(c == 1)
        def _():
            run(ib, ob)

    return gk(table_a, table_b, idx_a, idx_b)


def _scatter_pair(msg_l, idx_r, msg_r, idx_l, zeros):
    M = 2
    G = (_CH - 1) // M

    @functools.partial(
        pl.kernel,
        out_type=(jax.ShapeDtypeStruct((_N_PAD, _D), jnp.float32),
                  jax.ShapeDtypeStruct((_N_PAD, _D), jnp.float32)),
        mesh=_mesh(),
        scratch_types=[pltpu.VMEM((_CH, _CSZ), jnp.int32),
                       pltpu.VMEM_SHARED((_N_PAD, _D), jnp.float32)]
                      + [pltpu.VMEM((_CSZ, _D), jnp.float32)] * M
                      + [pltpu.SemaphoreType.DMA] * (2 * M),
    )
    def sk(ml, ir, mr, il, z, ol, orr, idx_v, acc, *bufs_and_sems):
        rows = bufs_and_sems[:M]
        rsem = bufs_and_sems[M:2 * M]
        ssem = bufs_and_sems[2 * M:]
        c = lax.axis_index("c")
        s = lax.axis_index("s")
        row0 = s * _RPT
        pltpu.sync_copy(z.at[pl.ds(row0, _RPT)], acc.at[pl.ds(row0, _RPT)])
        plsc.subcore_barrier()

        def run(msg, idx_hbm):
            base = s * _SLAB
            pltpu.sync_copy(idx_hbm.at[s], idx_v)
            for b in range(M):
                pltpu.async_copy(msg.at[pl.ds(base + b * _CSZ, _CSZ)],
                                 rows[b], rsem[b])

            def group(g, carry):
                for b in range(M):
                    j = g * M + b
                    pltpu.make_async_copy(msg.at[pl.ds(0, _CSZ)], rows[b],
                                          rsem[b]).wait()
                    pltpu.async_copy(rows[b], acc.at[idx_v.at[j]], ssem[b],
                                     add=True)

                @pl.when(g < G - 1)
                def _():
                    for b in range(M):
                        jn = (g + 1) * M + b
                        pltpu.make_async_copy(
                            rows[b], acc.at[idx_v.at[0]], ssem[b]).wait()
                        pltpu.async_copy(
                            msg.at[pl.ds(base + jn * _CSZ, _CSZ)],
                            rows[b], rsem[b])

                return carry

            lax.fori_loop(0, G, group, 0)
            for b in range(M):
                pltpu.make_async_copy(rows[b], acc.at[idx_v.at[0]],
                                      ssem[b]).wait()
            j = _CH - 1
            pltpu.async_copy(msg.at[pl.ds(base + j * _CSZ, _CSZ)], rows[0],
                             rsem[0])
            pltpu.make_async_copy(msg.at[pl.ds(0, _CSZ)], rows[0],
                                  rsem[0]).wait()
            pltpu.async_copy(rows[0], acc.at[idx_v.at[j]], ssem[0], add=True)
            pltpu.make_async_copy(rows[0], acc.at[idx_v.at[0]],
                                  ssem[0]).wait()

        @pl.when(c == 0)
        def _():
            run(ml, ir)

        @pl.when(c == 1)
        def _():
            run(mr, il)

        plsc.subcore_barrier()

        @pl.when(c == 0)
        def _():
            pltpu.sync_copy(acc.at[pl.ds(row0, _RPT)], ol.at[pl.ds(row0, _RPT)])

        @pl.when(c == 1)
        def _():
            pltpu.sync_copy(acc.at[pl.ds(row0, _RPT)], orr.at[pl.ds(row0, _RPT)])

    return sk(msg_l, idx_r, msg_r, idx_l, zeros)


def _ln_k(x, g, b):
    m = jnp.mean(x, -1, keepdims=True)
    d = x - m
    v = jnp.mean(d * d, -1, keepdims=True)
    return d * lax.rsqrt(v + 1e-5) * g + b


def _side(hb, hn, ex, W):
    (bw, nw, i0b, ig, ibb, i1w, i1b,
     gwb, gwn, gwe, g0b, gg, gbb, g1w, g1b) = [w[...] for w in W]
    h = jnp.dot(hb, bw) + jnp.dot(hn, nw) + i0b
    h = jnp.maximum(_ln_k(h, ig, ibb), 0.0)
    io = jnp.dot(h, i1w) + i1b
    g = (jnp.dot(hb, gwb) + jnp.dot(hn, gwn)
         + lax.dot_general(ex, gwe, (((0,), (0,)), ((), ()))) + g0b)
    g = jnp.maximum(_ln_k(g, gg, gbb), 0.0)
    go = jnp.dot(g, g1w) + g1b
    return io * jax.nn.sigmoid(go)


def _full_spec(w):
    return pl.BlockSpec(w.shape, lambda i, n=len(w.shape): (0,) * n)


def _edge_call(hb, hl, hr, ex, ws, off):
    BE = 6400
    grid = _H // BE

    def body(hb_r, hl_r, hr_r, ex_r, *rest):
        wrefs = rest[:30]
        ml_r, mr_r = rest[30:]
        hb_v = hb_r[...]
        ex_v = ex_r[...]
        ml_r[...] = _side(hb_v, hl_r[...], ex_v, wrefs[:15])
        mr_r[...] = _side(hb_v, hr_r[...], ex_v, wrefs[15:])

    obk = lambda w: pl.BlockSpec((BE, w), lambda i: (i + off, 0))
    blk = lambda w: pl.BlockSpec((BE, w), lambda i: (i, 0))
    ebk = pl.BlockSpec((_GD, BE), lambda i: (0, i + off))
    return pl.pallas_call(
        body,
        grid=(grid,),
        in_specs=[obk(_D), blk(_D), blk(_D), ebk] + [_full_spec(w) for w in ws],
        out_specs=[blk(_D), blk(_D)],
        out_shape=[jax.ShapeDtypeStruct((_H, _D), jnp.float32),
                   jax.ShapeDtypeStruct((_H, _D), jnp.float32)],
    )(hb, hl, hr, ex, *ws)


def _table_call(accs, hn, wml, wnl, bl, wmr, wnr, br):
    def body(al0_r, al1_r, ar0_r, ar1_r, hn_r, wml_r, wnl_r, bl_r,
             wmr_r, wnr_r, br_r, a_r, b_r):
        hn_v = hn_r[...]
        al = al0_r[...] + al1_r[...]
        ar = ar0_r[...] + ar1_r[...]
        a_r[...] = jnp.dot(al, wml_r[...]) + jnp.dot(hn_v, wnl_r[...]) + bl_r[...]
        b_r[...] = jnp.dot(ar, wmr_r[...]) + jnp.dot(hn_v, wnr_r[...]) + br_r[...]

    return pl.pallas_call(
        body,
        out_shape=[jax.ShapeDtypeStruct((_N_PAD, _D), jnp.float32),
                   jax.ShapeDtypeStruct((_N_PAD, _D), jnp.float32)],
    )(*accs, hn, wml, wnl, bl, wmr, wnr, br)


def _final_half(hb, al, br, ws, off, prev):
    BE = 1280
    grid = _H // BE

    def body(hb_r, al_r, br_r, wself, bself,
             og, obb, o1w, o1b, lng, lnb, *rest):
        out_r = rest[-1]
        hb_v = hb_r[...]
        h = al_r[...] + br_r[...] + jnp.dot(hb_v, wself[...]) + bself[...]
        h = jnp.maximum(_ln_k(h, og[...], obb[...]), 0.0)
        h = jnp.dot(h, o1w[...]) + o1b[...]
        out_r[...] = _ln_k(h + hb_v, lng[...], lnb[...])

    obk = pl.BlockSpec((BE, _D), lambda i: (i + off, 0))
    blk = pl.BlockSpec((BE, _D), lambda i: (i, 0))
    in_specs = [obk, blk, blk] + [_full_spec(w) for w in ws]
    args = [hb, al, br] + list(ws)
    kwargs = {}
    if prev is not None:
        in_specs.append(pl.BlockSpec(memory_space=pltpu.MemorySpace.HBM))
        args.append(prev)
        kwargs["input_output_aliases"] = {len(args) - 1: 0}
    return pl.pallas_call(
        body,
        grid=(grid,),
        in_specs=in_specs,
        out_specs=obk,
        out_shape=jax.ShapeDtypeStruct((_E, _D), jnp.float32),
        **kwargs,
    )(*args)


def _side_weights(p):
    it = p["inter"]
    g = p["gate"]
    gw = g["l0"]["w"]
    i0w = it["l0"]["w"]
    r = lambda x: x.reshape(1, -1)
    return [
        p["bond_w"] @ i0w, p["node_w"] @ i0w,
        r(it["l0"]["b"]), r(it["ln0"]["g"]), r(it["ln0"]["b"]),
        it["l1"]["w"], r(it["l1"]["b"]),
        gw[:_D], gw[_D:2 * _D], gw[2 * _D:],
        r(g["l0"]["b"]), r(g["ln0"]["g"]), r(g["ln0"]["b"]),
        g["l1"]["w"], r(g["l1"]["b"]),
    ]


def kernel(h_bond, bond_index, h_node, bond_extra, params):
    left = bond_index[0].astype(jnp.int32)
    right = bond_index[1].astype(jnp.int32)
    idx_l = left.reshape(2, _NH, _CH, _CSZ)
    idx_r = right.reshape(2, _NH, _CH, _CSZ)

    hn_p = jnp.pad(h_node, ((0, _N_PAD - _N), (0, 0)))
    zeros = jnp.zeros((_N_PAD, _D), jnp.float32)

    ws = _side_weights(params["ffn_l"]) + _side_weights(params["ffn_r"])
    hl0, hr0 = _gather_pair(hn_p, hn_p, idx_l[0], idx_r[0])
    hl1, hr1 = _gather_pair(hn_p, hn_p, idx_l[1], idx_r[1])
    ex_t = bond_extra.T
    ml0, mr0 = _edge_call(h_bond, hl0, hr0, ex_t, ws, 0)
    ml1, mr1 = _edge_call(h_bond, hl1, hr1, ex_t, ws, _H // 6400)
    accl0, accr0 = _scatter_pair(ml0, idx_r[0], mr0, idx_l[0], zeros)
    accl1, accr1 = _scatter_pair(ml1, idx_r[1], mr1, idx_l[1], zeros)

    r = lambda x: x.reshape(1, -1)
    fo = params["out"]
    o0w = fo["l0"]["w"]
    A, B = _table_call(
        (accl0, accl1, accr0, accr1), hn_p,
        params["msg_l"]["w"] @ o0w, params["node_l"]["w"] @ o0w,
        r((params["msg_l"]["b"] + params["node_l"]["b"]) @ o0w),
        params["msg_r"]["w"] @ o0w, params["node_r"]["w"] @ o0w,
        r((params["msg_r"]["b"] + params["node_r"]["b"]) @ o0w))

    al0, br0 = _gather_pair(A, B, idx_l[0], idx_r[0])
    al1, br1 = _gather_pair(A, B, idx_l[1], idx_r[1])

    fws = [params["self"]["w"] @ o0w,
           r(params["self"]["b"] @ o0w + fo["l0"]["b"]),
           r(fo["ln0"]["g"]), r(fo["ln0"]["b"]),
           fo["l1"]["w"], r(fo["l1"]["b"]),
           r(params["ln"]["g"]), r(params["ln"]["b"])]
    out0 = _final_half(h_bond, al0, br0, fws, 0, None)
    return _final_half(h_bond, al1, br1, fws, _H // 1280, out0)
